# baseline (speedup 1.0000x reference)
import numpy as np

import jax
import jax.numpy as jnp
from jax import lax
from jax.experimental import pallas as pl
from jax.experimental.pallas import tpu as pltpu
from jax.experimental.pallas import tpu_sc as plsc

N = 10000
E = 320000
H = 128
L = 4
B = 64
F_E = 12

NC = 2
NS = 16
NW = NC * NS
EPW = E // NW
CH = 80
NCHUNK = EPW // CH
NP = 10240
RPS = NP // NS
EB = 2000
NB = 10
NBR = N // NB

_P = (np.arange(H).reshape(H // 32, 2, 16).transpose(0, 2, 1).reshape(H // 32, 32)
      .reshape(H))


def _f32dot(a, b):
    return jnp.dot(a, b, preferred_element_type=jnp.float32)


def _ew_body(ea_ref, we_ref, be_ref, wmb_ref, bm_ref, o0, o1, o2, o3):
    e = jnp.maximum(_f32dot(ea_ref[...], we_ref[...]) + be_ref[...], 0.0)
    outs = (o0, o1, o2, o3)
    for l in range(L):
        outs[l][...] = (_f32dot(e, wmb_ref[l]) + bm_ref[l : l + 1, :]).astype(
            jnp.bfloat16
        )


def _ew_call(edge_attr, W_edge, b_edge2, Wm_bot, b_msg):
    return pl.pallas_call(
        _ew_body,
        grid=(E // EB,),
        in_specs=[
            pl.BlockSpec((EB, F_E), lambda i: (i, 0)),
            pl.BlockSpec((F_E, H), lambda i: (0, 0)),
            pl.BlockSpec((1, H), lambda i: (0, 0)),
            pl.BlockSpec((L, H, H), lambda i: (0, 0, 0)),
            pl.BlockSpec((L, H), lambda i: (0, 0)),
        ],
        out_specs=[pl.BlockSpec((EB, H), lambda i: (i, 0))] * L,
        out_shape=[jax.ShapeDtypeStruct((E, H), jnp.bfloat16)] * L,
    )(edge_attr, W_edge, b_edge2, Wm_bot, b_msg)


def _h0_body(x_ref, wn_ref, bn_ref, wmt_ref, h_ref, hw_ref):
    h = jnp.maximum(_f32dot(x_ref[...], wn_ref[...]) + bn_ref[...], 0.0)
    h_ref[...] = h
    hw_ref[...] = _f32dot(h, wmt_ref[...])


def _h0_call(x_n, W_node, b_node2, Wm_top0):
    fn = x_n.shape[1]
    return pl.pallas_call(
        _h0_body,
        grid=(NB,),
        in_specs=[
            pl.BlockSpec((NBR, fn), lambda i: (i, 0)),
            pl.BlockSpec((fn, H), lambda i: (0, 0)),
            pl.BlockSpec((1, H), lambda i: (0, 0)),
            pl.BlockSpec((H, H), lambda i: (0, 0)),
        ],
        out_specs=[pl.BlockSpec((NBR, H), lambda i: (i, 0))] * 2,
        out_shape=[jax.ShapeDtypeStruct((N, H), jnp.float32)] * 2,
    )(x_n, W_node, b_node2, Wm_top0)


def _upd_body(h_ref, agg_ref, wut_ref, wub_ref, bu_ref, wmt_ref, h_out, hw_out):
    agg = agg_ref[0] + agg_ref[1]
    hn = jnp.maximum(
        _f32dot(h_ref[...], wut_ref[...]) + _f32dot(agg, wub_ref[...]) + bu_ref[...],
        0.0,
    )
    h_out[...] = hn
    hw_out[...] = _f32dot(hn, wmt_ref[...])


def _upd_call(h, agg2, Wut, Wub, bu2, Wmt_next):
    return pl.pallas_call(
        _upd_body,
        grid=(NB,),
        in_specs=[
            pl.BlockSpec((NBR, H), lambda i: (i, 0)),
            pl.BlockSpec((NC, NBR, H), lambda i: (0, i, 0)),
            pl.BlockSpec((H, H), lambda i: (0, 0)),
            pl.BlockSpec((H, H), lambda i: (0, 0)),
            pl.BlockSpec((1, H), lambda i: (0, 0)),
            pl.BlockSpec((H, H), lambda i: (0, 0)),
        ],
        out_specs=[pl.BlockSpec((NBR, H), lambda i: (i, 0))] * 2,
        out_shape=[jax.ShapeDtypeStruct((N, H), jnp.float32)] * 2,
    )(h, agg2, Wut, Wub, bu2, Wmt_next)


def _fin_body(h_ref, agg_ref, wut_ref, wub_ref, bu_ref, batch_ref, h_out, pool_out):
    i = pl.program_id(0)
    agg = agg_ref[0] + agg_ref[1]
    hn = jnp.maximum(
        _f32dot(h_ref[...], wut_ref[...]) + _f32dot(agg, wub_ref[...]) + bu_ref[...],
        0.0,
    )
    h_out[...] = hn
    onehot = (batch_ref[0] == lax.broadcasted_iota(jnp.int32, (B, NBR), 0)).astype(
        jnp.float32
    )

    @pl.when(i == 0)
    def _():
        pool_out[...] = jnp.zeros_like(pool_out)

    pool_out[...] += _f32dot(onehot, hn)


def _fin_call(h, agg2, Wut, Wub, bu2, batch3):
    return pl.pallas_call(
        _fin_body,
        grid=(NB,),
        in_specs=[
            pl.BlockSpec((NBR, H), lambda i: (i, 0)),
            pl.BlockSpec((NC, NBR, H), lambda i: (0, i, 0)),
            pl.BlockSpec((H, H), lambda i: (0, 0)),
            pl.BlockSpec((H, H), lambda i: (0, 0)),
            pl.BlockSpec((1, H), lambda i: (0, 0)),
            pl.BlockSpec((1, 1, NBR), lambda i: (i, 0, 0)),
        ],
        out_specs=[
            pl.BlockSpec((NBR, H), lambda i: (i, 0)),
            pl.BlockSpec((B, H), lambda i: (0, 0)),
        ],
        out_shape=[
            jax.ShapeDtypeStruct((N, H), jnp.float32),
            jax.ShapeDtypeStruct((B, H), jnp.float32),
        ],
    )(h, agg2, Wut, Wub, bu2, batch3)


def _sc_edge_body(hw_hbm, ew_hbm, src_hbm, dst_hbm, out_hbm,
                  agg_s, ew0, ew1, gb0, gb1, rs,
                  sv0, sv1, sv2, sv3, dv0, dv1, dv2, dv3,
                  se0, se1, sg0, sg1, si0, si1, si2, si3):
    c = lax.axis_index("c")
    s = lax.axis_index("s")
    wid = s * NC + c
    base = wid * EPW
    zero16 = jnp.zeros((16,), jnp.float32)
    ewb = (ew0, ew1)
    gbb = (gb0, gb1)
    svv = (sv0, sv1, sv2, sv3)
    dvv = (dv0, dv1, dv2, dv3)
    sew = (se0, se1)
    sgb = (sg0, sg1)
    sid = (si0, si1, si2, si3)

    def zrow(i, carry):
        for f in range(H // 16):
            rs[i, pl.ds(f * 16, 16)] = zero16
        return carry

    lax.fori_loop(0, CH, zrow, 0)
    row0 = s * RPS
    for r in range(RPS // CH):
        pltpu.sync_copy(rs, agg_s.at[pl.ds(row0 + r * CH, CH)])
    plsc.subcore_barrier()

    def idx_issue(g, q):
        eb = base + g * CH
        pltpu.async_copy(src_hbm.at[pl.ds(eb, CH)], svv[q], sid[q])
        pltpu.async_copy(dst_hbm.at[pl.ds(eb, CH)], dvv[q], sid[q])

    def idx_wait(q):
        pltpu.make_async_copy(src_hbm.at[pl.ds(base, CH)], svv[q], sid[q]).wait()
        pltpu.make_async_copy(src_hbm.at[pl.ds(base, CH)], dvv[q], sid[q]).wait()

    hbase = wid * (EPW // 2)

    def data_issue(g, p, q):
        pltpu.async_copy(
            ew_hbm.at[pl.ds(hbase + g * (CH // 2), CH // 2)], ewb[p], sew[p]
        )
        pltpu.async_copy(hw_hbm.at[svv[q]], gbb[p], sgb[p])

    def data_wait(p):
        pltpu.make_async_copy(ew_hbm.at[pl.ds(hbase, CH // 2)], ewb[p], sew[p]).wait()
        pltpu.make_async_copy(hw_hbm.at[svv[0]], gbb[p], sgb[p]).wait()

    def compute_scatter(p, q):
        eb, gb = ewb[p], gbb[p]

        def row(i2, carry):
            for u in range(2):
                i = 2 * i2 + u
                for j in range(H // 32):
                    w = eb[i2, pl.ds(64 * u + 16 * j, 16)]
                    lo = lax.bitcast_convert_type(w << 16, jnp.float32)
                    hi = lax.bitcast_convert_type(w & jnp.int32(-65536), jnp.float32)
                    s1 = pl.ds(32 * j, 16)
                    s2 = pl.ds(32 * j + 16, 16)
                    rs[i, s1] = jnp.maximum(lo + gb[i, s1], 0.0)
                    rs[i, s2] = jnp.maximum(hi + gb[i, s2], 0.0)
            return carry

        lax.fori_loop(0, CH // 2, row, 0)
        pltpu.sync_copy(rs, agg_s.at[dvv[q]], add=True)

    for g in range(4):
        idx_issue(g, g)
    idx_wait(0)
    data_issue(0, 0, 0)
    idx_wait(1)
    data_issue(1, 1, 1)

    def pipe(t, carry):
        g0 = 4 * t
        for d in range(4):
            g = g0 + d
            p = d % 2
            q = d % 4

            @pl.when(g < NCHUNK)
            def _():
                data_wait(p)
                compute_scatter(p, q)

                @pl.when(g + 4 < NCHUNK)
                def _():
                    idx_issue(g + 4, q)

                @pl.when(g + 2 < NCHUNK)
                def _():
                    idx_wait((d + 2) % 4)
                    data_issue(g + 2, p, (d + 2) % 4)

        return carry

    lax.fori_loop(0, (NCHUNK + 3) // 4, pipe, 0)
    plsc.subcore_barrier()

    for r in range(RPS // CH):
        rr = row0 + r * CH
        pltpu.sync_copy(agg_s.at[pl.ds(rr, CH)], rs)
        pltpu.sync_copy(rs, out_hbm.at[c, pl.ds(rr, CH)])


def _make_sc_edge():
    mesh = plsc.VectorSubcoreMesh(core_axis_name="c", subcore_axis_name="s")
    bbuf = pltpu.VMEM((CH // 2, H), jnp.int32)
    fbuf = pltpu.VMEM((CH, H), jnp.float32)
    ibuf = pltpu.VMEM((CH,), jnp.int32)
    dma = pltpu.SemaphoreType.DMA
    return pl.kernel(
        _sc_edge_body,
        out_type=jax.ShapeDtypeStruct((NC, NP, H), jnp.float32),
        mesh=mesh,
        scratch_types=[pltpu.VMEM_SHARED((NP, H), jnp.float32)]
        + [bbuf] * 2 + [fbuf] * 3 + [ibuf] * 8 + [dma] * 8,
    )


def kernel(x_n, edge_index_n, edge_attr_n, x_n_batch, W_node, b_node,
           W_edge, b_edge, W_msg, b_msg, W_upd, b_upd):
    src = edge_index_n[0]
    dst = edge_index_n[1]
    Wm_top = W_msg[:, :H, :]
    Wm_bot = W_msg[:, H:, :]
    Wu_top = W_upd[:, :H, :]
    Wu_bot = W_upd[:, H:, :]
    b_node2 = b_node[None, :]
    b_edge2 = b_edge[None, :]
    batch3 = x_n_batch.reshape(NB, 1, NBR)

    perm = jnp.asarray(_P)
    ew = _ew_call(edge_attr_n, W_edge, b_edge2, Wm_bot[:, :, perm], b_msg[:, perm])
    ew = [
        lax.bitcast_convert_type(e.reshape(E, H // 2, 2), jnp.int32).reshape(E // 2, H)
        for e in ew
    ]
    h, hw = _h0_call(x_n, W_node, b_node2, Wm_top[0])
    sc_edge = _make_sc_edge()

    for l in range(L):
        agg2 = sc_edge(hw, ew[l], src, dst)
        bu2 = b_upd[l][None, :]
        if l < L - 1:
            h, hw = _upd_call(h, agg2, Wu_top[l], Wu_bot[l], bu2, Wm_top[l + 1])
        else:
            h, pooled = _fin_call(h, agg2, Wu_top[l], Wu_bot[l], bu2, batch3)
    return h, pooled


# speedup vs baseline: 4.0151x; 4.0151x over previous
import jax
import jax.numpy as jnp
from jax import lax
from jax.experimental import pallas as pl
from jax.experimental.pallas import tpu as pltpu
from jax.experimental.pallas import tpu_sc as plsc

N = 10000
E = 320000
H = 128
L = 4
B = 64
F_E = 12

NC = 2
NS = 16
NW = NC * NS
EPW = E // NW
CH = 80
NCHUNK = EPW // CH
NP = 10240
RPS = NP // NS
EB = 2000
NB = 10
NBR = N // NB

def _f32dot(a, b):
    return jnp.dot(a, b, preferred_element_type=jnp.float32)


def _ew_body(ea_ref, we_ref, be_ref, wmb_ref, bm_ref, o0, o1, o2, o3):
    e = jnp.maximum(_f32dot(ea_ref[...], we_ref[...]) + be_ref[...], 0.0)
    outs = (o0, o1, o2, o3)
    for l in range(L):
        outs[l][...] = _f32dot(e, wmb_ref[l]) + bm_ref[l : l + 1, :]


def _ew_call(edge_attr, W_edge, b_edge2, Wm_bot, b_msg):
    return pl.pallas_call(
        _ew_body,
        grid=(E // EB,),
        in_specs=[
            pl.BlockSpec((EB, F_E), lambda i: (i, 0)),
            pl.BlockSpec((F_E, H), lambda i: (0, 0)),
            pl.BlockSpec((1, H), lambda i: (0, 0)),
            pl.BlockSpec((L, H, H), lambda i: (0, 0, 0)),
            pl.BlockSpec((L, H), lambda i: (0, 0)),
        ],
        out_specs=[pl.BlockSpec((EB, H), lambda i: (i, 0))] * L,
        out_shape=[jax.ShapeDtypeStruct((E, H), jnp.float32)] * L,
    )(edge_attr, W_edge, b_edge2, Wm_bot, b_msg)


def _h0_body(x_ref, wn_ref, bn_ref, wmt_ref, h_ref, hw_ref):
    h = jnp.maximum(_f32dot(x_ref[...], wn_ref[...]) + bn_ref[...], 0.0)
    h_ref[...] = h
    hw_ref[...] = _f32dot(h, wmt_ref[...])


def _h0_call(x_n, W_node, b_node2, Wm_top0):
    fn = x_n.shape[1]
    return pl.pallas_call(
        _h0_body,
        grid=(NB,),
        in_specs=[
            pl.BlockSpec((NBR, fn), lambda i: (i, 0)),
            pl.BlockSpec((fn, H), lambda i: (0, 0)),
            pl.BlockSpec((1, H), lambda i: (0, 0)),
            pl.BlockSpec((H, H), lambda i: (0, 0)),
        ],
        out_specs=[pl.BlockSpec((NBR, H), lambda i: (i, 0))] * 2,
        out_shape=[jax.ShapeDtypeStruct((N, H), jnp.float32)] * 2,
    )(x_n, W_node, b_node2, Wm_top0)


def _upd_body(h_ref, agg_ref, wut_ref, wub_ref, bu_ref, wmt_ref, h_out, hw_out):
    agg = agg_ref[0] + agg_ref[1]
    hn = jnp.maximum(
        _f32dot(h_ref[...], wut_ref[...]) + _f32dot(agg, wub_ref[...]) + bu_ref[...],
        0.0,
    )
    h_out[...] = hn
    hw_out[...] = _f32dot(hn, wmt_ref[...])


def _upd_call(h, agg2, Wut, Wub, bu2, Wmt_next):
    return pl.pallas_call(
        _upd_body,
        grid=(NB,),
        in_specs=[
            pl.BlockSpec((NBR, H), lambda i: (i, 0)),
            pl.BlockSpec((NC, NBR, H), lambda i: (0, i, 0)),
            pl.BlockSpec((H, H), lambda i: (0, 0)),
            pl.BlockSpec((H, H), lambda i: (0, 0)),
            pl.BlockSpec((1, H), lambda i: (0, 0)),
            pl.BlockSpec((H, H), lambda i: (0, 0)),
        ],
        out_specs=[pl.BlockSpec((NBR, H), lambda i: (i, 0))] * 2,
        out_shape=[jax.ShapeDtypeStruct((N, H), jnp.float32)] * 2,
    )(h, agg2, Wut, Wub, bu2, Wmt_next)


def _fin_body(h_ref, agg_ref, wut_ref, wub_ref, bu_ref, batch_ref, h_out, pool_out):
    i = pl.program_id(0)
    agg = agg_ref[0] + agg_ref[1]
    hn = jnp.maximum(
        _f32dot(h_ref[...], wut_ref[...]) + _f32dot(agg, wub_ref[...]) + bu_ref[...],
        0.0,
    )
    h_out[...] = hn
    onehot = (batch_ref[0] == lax.broadcasted_iota(jnp.int32, (B, NBR), 0)).astype(
        jnp.float32
    )

    @pl.when(i == 0)
    def _():
        pool_out[...] = jnp.zeros_like(pool_out)

    pool_out[...] += _f32dot(onehot, hn)


def _fin_call(h, agg2, Wut, Wub, bu2, batch3):
    return pl.pallas_call(
        _fin_body,
        grid=(NB,),
        in_specs=[
            pl.BlockSpec((NBR, H), lambda i: (i, 0)),
            pl.BlockSpec((NC, NBR, H), lambda i: (0, i, 0)),
            pl.BlockSpec((H, H), lambda i: (0, 0)),
            pl.BlockSpec((H, H), lambda i: (0, 0)),
            pl.BlockSpec((1, H), lambda i: (0, 0)),
            pl.BlockSpec((1, 1, NBR), lambda i: (i, 0, 0)),
        ],
        out_specs=[
            pl.BlockSpec((NBR, H), lambda i: (i, 0)),
            pl.BlockSpec((B, H), lambda i: (0, 0)),
        ],
        out_shape=[
            jax.ShapeDtypeStruct((N, H), jnp.float32),
            jax.ShapeDtypeStruct((B, H), jnp.float32),
        ],
    )(h, agg2, Wut, Wub, bu2, batch3)


def _sc_edge_body(hw_hbm, ew_hbm, src_hbm, dst_hbm, out_hbm,
                  agg_s, ew0, ew1, gb0, gb1,
                  sv0, sv1, sv2, sv3, dv0, dv1, dv2, dv3,
                  se0, se1, sg0, sg1, si0, si1, si2, si3):
    c = lax.axis_index("c")
    s = lax.axis_index("s")
    wid = s * NC + c
    base = wid * EPW
    zero16 = jnp.zeros((16,), jnp.float32)
    ewb = (ew0, ew1)
    gbb = (gb0, gb1)
    svv = (sv0, sv1, sv2, sv3)
    dvv = (dv0, dv1, dv2, dv3)
    sew = (se0, se1)
    sgb = (sg0, sg1)
    sid = (si0, si1, si2, si3)

    def zrow(i, carry):
        for f in range(H // 16):
            ew0[i, pl.ds(f * 16, 16)] = zero16
        return carry

    lax.fori_loop(0, CH, zrow, 0)
    row0 = s * RPS
    for r in range(RPS // CH):
        pltpu.sync_copy(ew0, agg_s.at[pl.ds(row0 + r * CH, CH)])
    plsc.subcore_barrier()

    def idx_issue(g, q):
        eb = base + g * CH
        pltpu.async_copy(src_hbm.at[pl.ds(eb, CH)], svv[q], sid[q])
        pltpu.async_copy(dst_hbm.at[pl.ds(eb, CH)], dvv[q], sid[q])

    def idx_wait(q):
        pltpu.make_async_copy(src_hbm.at[pl.ds(base, CH)], svv[q], sid[q]).wait()
        pltpu.make_async_copy(src_hbm.at[pl.ds(base, CH)], dvv[q], sid[q]).wait()

    def data_issue(g, p, q):
        pltpu.async_copy(ew_hbm.at[pl.ds(base + g * CH, CH)], ewb[p], sew[p])
        pltpu.async_copy(hw_hbm.at[svv[q]], gbb[p], sgb[p])

    def data_wait(p):
        pltpu.make_async_copy(ew_hbm.at[pl.ds(base, CH)], ewb[p], sew[p]).wait()
        pltpu.make_async_copy(ew_hbm.at[pl.ds(base, CH)], gbb[p], sgb[p]).wait()

    def compute_scatter(p, q):
        eb, gb = ewb[p], gbb[p]

        def row(i, carry):
            for u in range(2):
                for f in range(H // 16):
                    sl = pl.ds(f * 16, 16)
                    gb[2 * i + u, sl] = jnp.maximum(
                        eb[2 * i + u, sl] + gb[2 * i + u, sl], 0.0
                    )
            return carry

        lax.fori_loop(0, CH // 2, row, 0)
        pltpu.sync_copy(gb, agg_s.at[dvv[q]], add=True)

    for g in range(4):
        idx_issue(g, g)
    idx_wait(0)
    data_issue(0, 0, 0)
    idx_wait(1)
    data_issue(1, 1, 1)

    def pipe(t, carry):
        g0 = 4 * t
        for d in range(4):
            g = g0 + d
            p = d % 2
            q = d % 4

            @pl.when(g < NCHUNK)
            def _():
                data_wait(p)
                compute_scatter(p, q)

                @pl.when(g + 4 < NCHUNK)
                def _():
                    idx_issue(g + 4, q)

                @pl.when(g + 2 < NCHUNK)
                def _():
                    idx_wait((d + 2) % 4)
                    data_issue(g + 2, p, (d + 2) % 4)

        return carry

    lax.fori_loop(0, (NCHUNK + 3) // 4, pipe, 0)
    plsc.subcore_barrier()

    for r in range(RPS // CH):
        rr = row0 + r * CH
        pltpu.sync_copy(agg_s.at[pl.ds(rr, CH)], ew0)
        pltpu.sync_copy(ew0, out_hbm.at[c, pl.ds(rr, CH)])


def _make_sc_edge():
    mesh = plsc.VectorSubcoreMesh(core_axis_name="c", subcore_axis_name="s")
    fbuf = pltpu.VMEM((CH, H), jnp.float32)
    ibuf = pltpu.VMEM((CH,), jnp.int32)
    dma = pltpu.SemaphoreType.DMA
    return pl.kernel(
        _sc_edge_body,
        out_type=jax.ShapeDtypeStruct((NC, NP, H), jnp.float32),
        mesh=mesh,
        scratch_types=[pltpu.VMEM_SHARED((NP, H), jnp.float32)]
        + [fbuf] * 4 + [ibuf] * 8 + [dma] * 8,
    )


def kernel(x_n, edge_index_n, edge_attr_n, x_n_batch, W_node, b_node,
           W_edge, b_edge, W_msg, b_msg, W_upd, b_upd):
    src = edge_index_n[0]
    dst = edge_index_n[1]
    Wm_top = W_msg[:, :H, :]
    Wm_bot = W_msg[:, H:, :]
    Wu_top = W_upd[:, :H, :]
    Wu_bot = W_upd[:, H:, :]
    b_node2 = b_node[None, :]
    b_edge2 = b_edge[None, :]
    batch3 = x_n_batch.reshape(NB, 1, NBR)

    ew = _ew_call(edge_attr_n, W_edge, b_edge2, Wm_bot, b_msg)
    h, hw = _h0_call(x_n, W_node, b_node2, Wm_top[0])
    sc_edge = _make_sc_edge()

    for l in range(L):
        agg2 = sc_edge(hw, ew[l], src, dst)
        bu2 = b_upd[l][None, :]
        if l < L - 1:
            h, hw = _upd_call(h, agg2, Wu_top[l], Wu_bot[l], bu2, Wm_top[l + 1])
        else:
            h, pooled = _fin_call(h, agg2, Wu_top[l], Wu_bot[l], bu2, batch3)
    return h, pooled
